# Initial kernel scaffold; baseline (speedup 1.0000x reference)
import jax
import jax.numpy as jnp
from jax.experimental import pallas as pl


def kernel(x, labels, running_mean, running_var, class_count):
    raise NotImplementedError("write your pallas kernel here")


# speedup vs baseline: 3.2299x; 3.2299x over previous
import jax
import jax.numpy as jnp
from jax import lax
from jax.experimental import pallas as pl
from jax.experimental.pallas import tpu as pltpu

_C = 1000
_B = 4096
_D = 2048
_GD = 8
_GB = 8
_BD = _D // _GD
_BB = _B // _GB


def _body(lab_ref, x_ref, rm_ref, rv_ref, cc_ref,
          nm_ref, nv_ref, ncnt_ref, reg_ref,
          sum_ref, sum2_ref, n_ref):
    d = pl.program_id(0)
    b = pl.program_id(1)

    @pl.when(jnp.logical_and(d == 0, b == 0))
    def _():
        reg_ref[...] = jnp.zeros_like(reg_ref)

    @pl.when(b == 0)
    def _():
        sum_ref[...] = jnp.zeros_like(sum_ref)
        sum2_ref[...] = jnp.zeros_like(sum2_ref)
        n_ref[...] = jnp.zeros_like(n_ref)

    labels = lab_ref[0, 0, :]
    oh = (labels[:, None] ==
          lax.broadcasted_iota(jnp.int32, (_BB, _C), 1)).astype(jnp.float32)
    xb = x_ref[...]
    dn = (((0,), (0,)), ((), ()))
    sum_ref[...] += lax.dot_general(oh, xb, dn,
                                    preferred_element_type=jnp.float32)
    sum2_ref[...] += lax.dot_general(oh, xb * xb, dn,
                                     preferred_element_type=jnp.float32)
    n_ref[...] += jnp.sum(oh, axis=0)[:, None]

    @pl.when(b == _GB - 1)
    def _():
        cc = cc_ref[...].astype(jnp.float32)
        n = n_ref[...]
        ccn = cc + n
        pos = ccn > 0
        denom = jnp.where(pos, ccn, 1.0)
        rm = rm_ref[...]
        rv = rv_ref[...]
        s = sum_ref[...]
        s2 = sum2_ref[...]
        nm_ref[...] = jnp.where(pos, (rm * cc + s) / denom, rm)
        nv_ref[...] = jnp.where(pos, (rv * cc + s2) / denom, rv)

        @pl.when(d == 0)
        def _():
            ncnt_ref[...] = ccn.astype(jnp.int32)

        t1 = jnp.sum(s2)
        t2 = jnp.sum(s * rm)
        t3 = jnp.sum(n * (rm * rm))
        reg_ref[...] += jnp.full((1, 1), t1 - 2.0 * t2 + t3)

        @pl.when(d == _GD - 1)
        def _():
            reg_ref[...] = jnp.sqrt(reg_ref[...])


def kernel(x, labels, running_mean, running_var, class_count):
    labels3 = labels.reshape(_GB, 1, _BB)
    out_shapes = (
        jax.ShapeDtypeStruct((_C, _D), jnp.float32),
        jax.ShapeDtypeStruct((_C, _D), jnp.float32),
        jax.ShapeDtypeStruct((_C, 1), jnp.int32),
        jax.ShapeDtypeStruct((1, 1), jnp.float32),
    )
    grid = (_GD, _GB)
    nm, nv, ncnt, reg = pl.pallas_call(
        _body,
        grid=grid,
        in_specs=[
            pl.BlockSpec((1, 1, _BB), lambda d, b: (b, 0, 0)),
            pl.BlockSpec((_BB, _BD), lambda d, b: (b, d)),
            pl.BlockSpec((_C, _BD), lambda d, b: (0, d)),
            pl.BlockSpec((_C, _BD), lambda d, b: (0, d)),
            pl.BlockSpec((_C, 1), lambda d, b: (0, 0)),
        ],
        out_specs=(
            pl.BlockSpec((_C, _BD), lambda d, b: (0, d)),
            pl.BlockSpec((_C, _BD), lambda d, b: (0, d)),
            pl.BlockSpec((_C, 1), lambda d, b: (0, 0)),
            pl.BlockSpec((1, 1), lambda d, b: (0, 0)),
        ),
        out_shape=out_shapes,
        scratch_shapes=[
            pltpu.VMEM((_C, _BD), jnp.float32),
            pltpu.VMEM((_C, _BD), jnp.float32),
            pltpu.VMEM((_C, 1), jnp.float32),
        ],
    )(labels3, x, running_mean, running_var, class_count)
    return nm, nv, ncnt, reg.reshape(())
